# Initial kernel scaffold; baseline (speedup 1.0000x reference)
import functools

import jax
import jax.numpy as jnp
from jax import lax
from jax.experimental import pallas as pl
from jax.experimental.pallas import tpu as pltpu
from jax.experimental.pallas import tpu_sc as plsc

N_NODES = 10000
N_EDGES = 320000
D = 128
SLOPE = 0.01

NC = 2
NS = 16
NW = NC * NS
EPW = N_EDGES // NW
CH = 80
NCHUNK = EPW // CH
RPS = N_NODES // NS
ZR = 125
DEGW = 16


def _sc_accumulate(src, dst, x):
    mesh = plsc.VectorSubcoreMesh(core_axis_name="c", subcore_axis_name="s")

    @functools.partial(
        pl.kernel,
        mesh=mesh,
        out_type=[
            jax.ShapeDtypeStruct((NC, N_NODES, D), jnp.float32),
            jax.ShapeDtypeStruct((NC, N_NODES, DEGW), jnp.float32),
        ],
        scratch_types=[
            pltpu.VMEM((CH,), jnp.int32),
            pltpu.VMEM((CH,), jnp.int32),
            pltpu.VMEM((CH, D), jnp.float32),
            pltpu.VMEM((CH, DEGW), jnp.float32),
            pltpu.VMEM((ZR, D), jnp.float32),
            pltpu.VMEM((ZR, DEGW), jnp.float32),
            pltpu.VMEM_SHARED((N_NODES, D), jnp.float32),
            pltpu.VMEM_SHARED((N_NODES, DEGW), jnp.float32),
        ],
    )
    def acc_kernel(src_hbm, dst_hbm, x_hbm, sum_hbm, deg_hbm,
                   src_v, dst_v, rows_v, ones_v, zrow_v, zdeg_v,
                   acc_sh, deg_sh):
        cid = lax.axis_index("c")
        sid = lax.axis_index("s")
        wid = cid * NS + sid

        @pl.loop(0, ZR)
        def _(i):
            zdeg_v[i, :] = jnp.zeros((DEGW,), jnp.float32)

            @pl.loop(0, D, step=16)
            def _(j):
                zrow_v[i, pl.ds(j, 16)] = jnp.zeros((16,), jnp.float32)

        @pl.loop(0, CH)
        def _(i):
            ones_v[i, :] = jnp.ones((DEGW,), jnp.float32)

        @pl.loop(0, RPS // ZR)
        def _(k):
            r = sid * RPS + k * ZR
            pltpu.sync_copy(zrow_v, acc_sh.at[pl.ds(r, ZR)])
            pltpu.sync_copy(zdeg_v, deg_sh.at[pl.ds(r, ZR)])

        plsc.subcore_barrier()

        base = wid * EPW

        @pl.loop(0, NCHUNK)
        def _(k):
            off = base + k * CH
            pltpu.sync_copy(src_hbm.at[pl.ds(off, CH)], src_v)
            pltpu.sync_copy(dst_hbm.at[pl.ds(off, CH)], dst_v)
            pltpu.sync_copy(x_hbm.at[src_v], rows_v)
            pltpu.sync_copy(rows_v, acc_sh.at[dst_v], add=True)
            pltpu.sync_copy(ones_v, deg_sh.at[dst_v], add=True)

        plsc.subcore_barrier()

        @pl.loop(0, RPS // ZR)
        def _(k):
            r = sid * RPS + k * ZR
            pltpu.sync_copy(acc_sh.at[pl.ds(r, ZR)],
                            sum_hbm.at[cid, pl.ds(r, ZR)])
            pltpu.sync_copy(deg_sh.at[pl.ds(r, ZR)],
                            deg_hbm.at[cid, pl.ds(r, ZR)])

    return acc_kernel(src, dst, x)


def _tc_body(sum_ref, deg_ref, x_ref, wl_ref, bl_ref, wr_ref, o_ref):
    s = sum_ref[0] + sum_ref[1]
    deg = deg_ref[0, :, 0] + deg_ref[1, :, 0]
    aggr = s / jnp.clip(deg, 1.0, None)[:, None]
    out = (
        lax.dot_general(aggr, wl_ref[...], (((1,), (1,)), ((), ())),
                        preferred_element_type=jnp.float32)
        + bl_ref[...]
        + lax.dot_general(x_ref[...], wr_ref[...], (((1,), (1,)), ((), ())),
                          preferred_element_type=jnp.float32)
    )
    o_ref[...] = jnp.where(out >= 0, out, SLOPE * out)


def _tc_epilogue(sums, degs, x, W_l, b_l, W_r):
    BN = 2000
    return pl.pallas_call(
        _tc_body,
        grid=(N_NODES // BN,),
        in_specs=[
            pl.BlockSpec((NC, BN, D), lambda i: (0, i, 0)),
            pl.BlockSpec((NC, BN, DEGW), lambda i: (0, i, 0)),
            pl.BlockSpec((BN, D), lambda i: (i, 0)),
            pl.BlockSpec((D, D), lambda i: (0, 0)),
            pl.BlockSpec((1, D), lambda i: (0, 0)),
            pl.BlockSpec((D, D), lambda i: (0, 0)),
        ],
        out_specs=pl.BlockSpec((BN, D), lambda i: (i, 0)),
        out_shape=jax.ShapeDtypeStruct((N_NODES, D), jnp.float32),
    )(sums, degs, x, W_l, b_l, W_r)


@jax.jit
def kernel(x, edge_index, W_l, b_l, W_r):
    src = edge_index[0]
    dst = edge_index[1]
    sums, degs = _sc_accumulate(src, dst, x)
    return _tc_epilogue(sums, degs, x, W_l, b_l.reshape(1, D), W_r)


# speedup vs baseline: 4.9071x; 4.9071x over previous
import functools

import jax
import jax.numpy as jnp
from jax import lax
from jax.experimental import pallas as pl
from jax.experimental.pallas import tpu as pltpu
from jax.experimental.pallas import tpu_sc as plsc

N_NODES = 10000
N_EDGES = 320000
D = 128
SLOPE = 0.01

NC = 2
NS = 16
NW = NC * NS
EPW = N_EDGES // NW
CH = 80
NCHUNK = EPW // CH
NZCH = N_NODES // CH


def _sc_accumulate(src, dst, x):
    mesh = plsc.VectorSubcoreMesh(core_axis_name="c", subcore_axis_name="s")

    @functools.partial(
        pl.kernel,
        mesh=mesh,
        out_type=[
            jax.ShapeDtypeStruct((NC, N_NODES, D), jnp.float32),
            jax.ShapeDtypeStruct((NC, N_NODES, D), jnp.float32),
        ],
        scratch_types=[
            pltpu.VMEM((CH,), jnp.int32),
            pltpu.VMEM((CH,), jnp.int32),
            pltpu.VMEM((CH,), jnp.int32),
            pltpu.VMEM((CH, D), jnp.float32),
            pltpu.VMEM((CH, D), jnp.float32),
            pltpu.VMEM_SHARED((N_NODES, D), jnp.float32),
        ],
    )
    def acc_kernel(src_hbm, dst_hbm, x_hbm, sum_hbm, cnt_hbm,
                   src_v, dst_v, idx_v, rows_v, ones_v, acc_sh):
        cid = lax.axis_index("c")
        sid = lax.axis_index("s")
        wid = cid * NS + sid
        base = wid * EPW

        def fill(buf, value):
            @pl.loop(0, CH)
            def _(i):
                @pl.loop(0, D, step=16)
                def _(j):
                    buf[i, pl.ds(j, 16)] = jnp.full((16,), value, jnp.float32)

        def zero_acc():
            @pl.loop(sid, NZCH, step=NS)
            def _(k):
                r = k * CH

                @pl.loop(0, CH, step=16)
                def _(j):
                    idx_v[pl.ds(j, 16)] = (
                        jnp.arange(16, dtype=jnp.int32) + (r + j))

                pltpu.sync_copy(rows_v, acc_sh.at[idx_v])

        def write_out(dst_hbm_ref):
            @pl.loop(sid, NZCH, step=NS)
            def _(k):
                r = k * CH

                @pl.loop(0, CH, step=16)
                def _(j):
                    idx_v[pl.ds(j, 16)] = (
                        jnp.arange(16, dtype=jnp.int32) + (r + j))

                pltpu.sync_copy(acc_sh.at[idx_v], rows_v)
                pltpu.sync_copy(rows_v, dst_hbm_ref.at[cid, pl.ds(r, CH)])

        fill(rows_v, 0.0)
        fill(ones_v, 1.0)
        zero_acc()
        plsc.subcore_barrier()

        @pl.loop(0, NCHUNK)
        def _(k):
            off = base + k * CH
            pltpu.sync_copy(src_hbm.at[pl.ds(off, CH)], src_v)
            pltpu.sync_copy(x_hbm.at[src_v], rows_v)
            pltpu.sync_copy(dst_hbm.at[pl.ds(off, CH)], dst_v)
            pltpu.sync_copy(rows_v, acc_sh.at[dst_v], add=True)

        plsc.subcore_barrier()
        write_out(sum_hbm)
        plsc.subcore_barrier()

        fill(rows_v, 0.0)
        zero_acc()
        plsc.subcore_barrier()

        @pl.loop(0, NCHUNK)
        def _(k):
            off = base + k * CH
            pltpu.sync_copy(dst_hbm.at[pl.ds(off, CH)], dst_v)
            pltpu.sync_copy(ones_v, acc_sh.at[dst_v], add=True)

        plsc.subcore_barrier()
        write_out(cnt_hbm)

    return acc_kernel(src, dst, x)


def _tc_body(sum_ref, cnt_ref, x_ref, wl_ref, bl_ref, wr_ref, o_ref):
    s = sum_ref[0] + sum_ref[1]
    deg = cnt_ref[0, :, 0] + cnt_ref[1, :, 0]
    aggr = s / jnp.clip(deg, 1.0, None)[:, None]
    out = (
        lax.dot_general(aggr, wl_ref[...], (((1,), (1,)), ((), ())),
                        preferred_element_type=jnp.float32)
        + bl_ref[...]
        + lax.dot_general(x_ref[...], wr_ref[...], (((1,), (1,)), ((), ())),
                          preferred_element_type=jnp.float32)
    )
    o_ref[...] = jnp.where(out >= 0, out, SLOPE * out)


def _tc_epilogue(sums, cnts, x, W_l, b_l, W_r):
    BN = 2000
    return pl.pallas_call(
        _tc_body,
        grid=(N_NODES // BN,),
        in_specs=[
            pl.BlockSpec((NC, BN, D), lambda i: (0, i, 0)),
            pl.BlockSpec((NC, BN, D), lambda i: (0, i, 0)),
            pl.BlockSpec((BN, D), lambda i: (i, 0)),
            pl.BlockSpec((D, D), lambda i: (0, 0)),
            pl.BlockSpec((1, D), lambda i: (0, 0)),
            pl.BlockSpec((D, D), lambda i: (0, 0)),
        ],
        out_specs=pl.BlockSpec((BN, D), lambda i: (i, 0)),
        out_shape=jax.ShapeDtypeStruct((N_NODES, D), jnp.float32),
    )(sums, cnts, x, W_l, b_l, W_r)


@jax.jit
def kernel(x, edge_index, W_l, b_l, W_r):
    src = edge_index[0]
    dst = edge_index[1]
    sums, cnts = _sc_accumulate(src, dst, x)
    return _tc_epilogue(sums, cnts, x, W_l, b_l.reshape(1, D), W_r)


# speedup vs baseline: 6.8997x; 1.4061x over previous
import functools

import jax
import jax.numpy as jnp
from jax import lax
from jax.experimental import pallas as pl
from jax.experimental.pallas import tpu as pltpu
from jax.experimental.pallas import tpu_sc as plsc

N_NODES = 10000
N_EDGES = 320000
D = 128
SLOPE = 0.01

NC = 2
NS = 16
NW = NC * NS
EPW = N_EDGES // NW
CH = 80
NCHUNK = EPW // CH
NZCH = N_NODES // CH
NIS = 4
NMAIN = (NCHUNK - 1) // NIS * NIS


def _sc_accumulate(eidx, x):
    mesh = plsc.VectorSubcoreMesh(core_axis_name="c", subcore_axis_name="s")

    @functools.partial(
        pl.kernel,
        mesh=mesh,
        out_type=[
            jax.ShapeDtypeStruct((NC, N_NODES, D), jnp.float32),
            jax.ShapeDtypeStruct((NC, N_NODES, D), jnp.float32),
        ],
        scratch_types=[
            pltpu.VMEM((NIS, 2, CH), jnp.int32),
            pltpu.VMEM((2, CH), jnp.int32),
            pltpu.VMEM((CH,), jnp.int32),
            pltpu.VMEM((2, CH, D), jnp.float32),
            pltpu.VMEM_SHARED((N_NODES, D), jnp.float32),
            pltpu.SemaphoreType.DMA,
            pltpu.SemaphoreType.DMA,
            pltpu.SemaphoreType.DMA,
            pltpu.SemaphoreType.DMA,
            pltpu.SemaphoreType.DMA,
            pltpu.SemaphoreType.DMA,
            pltpu.SemaphoreType.DMA,
        ],
    )
    def acc_kernel(eidx_hbm, x_hbm, sum_hbm, cnt_hbm,
                   eblk, dblk, idx_v, rows, acc_sh,
                   isem0, isem1, isem2, isem3, gsem0, gsem1, ssem):
        cid = lax.axis_index("c")
        sid = lax.axis_index("s")
        wid = cid * NS + sid
        cbase = wid * NCHUNK
        isems = (isem0, isem1, isem2, isem3)
        gsems = (gsem0, gsem1)

        def fill(buf, value):
            @pl.loop(0, CH)
            def _(i):
                @pl.loop(0, D, step=16)
                def _(j):
                    buf[i, pl.ds(j, 16)] = jnp.full((16,), value, jnp.float32)

        def fill_idx_v(r):
            @pl.loop(0, CH, step=16)
            def _(j):
                idx_v[pl.ds(j, 16)] = jnp.arange(16, dtype=jnp.int32) + (r + j)

        def zero_acc(zsrc):
            @pl.loop(sid, NZCH, step=NS)
            def _(k):
                fill_idx_v(k * CH)
                pltpu.sync_copy(zsrc, acc_sh.at[idx_v])

        def write_out(dst_hbm_ref):
            @pl.loop(sid, NZCH, step=NS)
            def _(k):
                r = k * CH
                fill_idx_v(r)
                pltpu.sync_copy(acc_sh.at[idx_v], rows.at[1])
                pltpu.sync_copy(rows.at[1], dst_hbm_ref.at[cid, pl.ds(r, CH)])

        def fetch_idx(c, islot):
            pltpu.async_copy(eidx_hbm.at[cbase + c], eblk.at[islot],
                             isems[islot])

        def wait_idx(islot):
            pltpu.make_async_copy(eidx_hbm.at[0], eblk.at[islot],
                                  isems[islot]).wait()

        def wait_scatter():
            pltpu.make_async_copy(rows.at[0], acc_sh.at[eblk.at[0, 1]],
                                  ssem).wait()

        def prime(zsrc):
            for q in range(NIS):
                fetch_idx(q, q)
            fill_idx_v(0)
            pltpu.async_copy(zsrc, acc_sh.at[idx_v], ssem, add=True)

        def drain_tail():
            wait_scatter()
            wait_idx((NCHUNK + 1) % NIS)

        fill(rows.at[0], 0.0)
        fill(rows.at[1], 0.0)
        zero_acc(rows.at[0])
        plsc.subcore_barrier()

        def p1_slot(c, q):
            par = q % 2
            islot = q % NIS
            wait_idx(islot)
            g = pltpu.async_copy(x_hbm.at[eblk.at[islot, 0]], rows.at[par],
                                 gsems[par])
            fetch_idx(c + 2, (q + 2) % NIS)
            g.wait()
            wait_scatter()
            pltpu.async_copy(rows.at[par], acc_sh.at[eblk.at[islot, 1]],
                             ssem, add=True)

        prime(rows.at[0])

        @pl.loop(0, NMAIN, step=NIS)
        def _(k):
            for q in range(NIS):
                p1_slot(k + q, q)

        wait_idx((NCHUNK - 1) % NIS)
        g = pltpu.async_copy(x_hbm.at[eblk.at[(NCHUNK - 1) % NIS, 0]],
                             rows.at[0], gsems[0])
        g.wait()
        wait_scatter()
        pltpu.async_copy(rows.at[0], acc_sh.at[eblk.at[(NCHUNK - 1) % NIS, 1]],
                         ssem, add=True)
        drain_tail()

        plsc.subcore_barrier()
        write_out(sum_hbm)
        plsc.subcore_barrier()

        fill(rows.at[0], 1.0)
        fill(rows.at[1], 0.0)
        zero_acc(rows.at[1])
        plsc.subcore_barrier()

        @pl.loop(0, NCHUNK)
        def _(k):
            pltpu.sync_copy(eidx_hbm.at[cbase + k], dblk)
            pltpu.sync_copy(rows.at[0], acc_sh.at[dblk.at[1]], add=True)

        plsc.subcore_barrier()
        write_out(cnt_hbm)

    return acc_kernel(eidx, x)


def _tc_body(sum_ref, cnt_ref, x_ref, wl_ref, bl_ref, wr_ref, o_ref):
    s = sum_ref[0] + sum_ref[1]
    deg = cnt_ref[0, :, 0] + cnt_ref[1, :, 0]
    aggr = s / jnp.clip(deg, 1.0, None)[:, None]
    out = (
        lax.dot_general(aggr, wl_ref[...], (((1,), (1,)), ((), ())),
                        preferred_element_type=jnp.float32)
        + bl_ref[...]
        + lax.dot_general(x_ref[...], wr_ref[...], (((1,), (1,)), ((), ())),
                          preferred_element_type=jnp.float32)
    )
    o_ref[...] = jnp.where(out >= 0, out, SLOPE * out)


def _tc_epilogue(sums, cnts, x, W_l, b_l, W_r):
    BN = 2000
    return pl.pallas_call(
        _tc_body,
        grid=(N_NODES // BN,),
        in_specs=[
            pl.BlockSpec((NC, BN, D), lambda i: (0, i, 0)),
            pl.BlockSpec((NC, BN, D), lambda i: (0, i, 0)),
            pl.BlockSpec((BN, D), lambda i: (i, 0)),
            pl.BlockSpec((D, D), lambda i: (0, 0)),
            pl.BlockSpec((1, D), lambda i: (0, 0)),
            pl.BlockSpec((D, D), lambda i: (0, 0)),
        ],
        out_specs=pl.BlockSpec((BN, D), lambda i: (i, 0)),
        out_shape=jax.ShapeDtypeStruct((N_NODES, D), jnp.float32),
    )(sums, cnts, x, W_l, b_l, W_r)


@jax.jit
def kernel(x, edge_index, W_l, b_l, W_r):
    src = edge_index[0].reshape(NW * NCHUNK, 1, CH)
    dst = edge_index[1].reshape(NW * NCHUNK, 1, CH)
    eidx = jnp.concatenate([src, dst], axis=1)
    eidx = jnp.concatenate(
        [eidx, jnp.zeros((8, 2, CH), jnp.int32)], axis=0)
    sums, cnts = _sc_accumulate(eidx, x)
    return _tc_epilogue(sums, cnts, x, W_l, b_l.reshape(1, D), W_r)
